# Initial kernel scaffold; baseline (speedup 1.0000x reference)
import jax
import jax.numpy as jnp
from jax.experimental import pallas as pl


def kernel(x, edge_index, edge_attr, W1, aS1, aD1, We1, aE1, b1, g1, be1, W2, aS2, aD2, We2, aE2, b2, g2, be2):
    raise NotImplementedError("write your pallas kernel here")


# speedup vs baseline: 1.1988x; 1.1988x over previous
import functools
import jax
import jax.numpy as jnp
from jax.experimental import pallas as pl

_N = 10000
_E = 160000
_HID = 256
_HEADS = 4


def _ln_relu_body(h_ref, g_ref, b_ref, o_ref, *, relu):
    h = h_ref[...]
    m = jnp.mean(h, axis=-1, keepdims=True)
    v = jnp.mean((h - m) ** 2, axis=-1, keepdims=True)
    y = (h - m) * jax.lax.rsqrt(v + 1e-5) * g_ref[...] + b_ref[...]
    if relu:
        y = jnp.maximum(y, 0.0)
    o_ref[...] = y


def _ln(h, g, b, relu):
    n, d = h.shape
    blk = 1000
    return pl.pallas_call(
        functools.partial(_ln_relu_body, relu=relu),
        out_shape=jax.ShapeDtypeStruct((n, d), jnp.float32),
        grid=(n // blk,),
        in_specs=[
            pl.BlockSpec((blk, d), lambda i: (i, 0)),
            pl.BlockSpec((1, d), lambda i: (0, 0)),
            pl.BlockSpec((1, d), lambda i: (0, 0)),
        ],
        out_specs=pl.BlockSpec((blk, d), lambda i: (i, 0)),
    )(h, g.reshape(1, d), b.reshape(1, d))


def _gat_layer(x, src, dst, ea, W, aS, aD, We, aE, bias, H, C):
    N = x.shape[0]
    h = (x @ W).reshape(N, H, C)
    a_src = (h * aS).sum(-1)
    a_dst = (h * aD).sum(-1)
    V = (We.reshape(We.shape[0], H, C) * aE).sum(-1)
    a_edge = ea @ V

    ones = jnp.ones((ea.shape[0],), jnp.float32)
    cnt = jax.ops.segment_sum(ones, dst, num_segments=N)
    ae_sum = jax.ops.segment_sum(a_edge, dst, num_segments=N)
    mean_ae = ae_sum / jnp.clip(cnt, 1.0)[:, None]

    alpha = jax.nn.leaky_relu(a_src[src] + a_dst[dst] + a_edge, 0.2)
    ex = jnp.exp(alpha)
    alpha_loop = jax.nn.leaky_relu(a_src + a_dst + mean_ae, 0.2)
    ex_loop = jnp.exp(alpha_loop)

    den = jax.ops.segment_sum(ex, dst, num_segments=N) + ex_loop
    coef = ex / (den[dst] + 1e-16)
    coef_loop = ex_loop / (den + 1e-16)

    out = jax.ops.segment_sum(h[src] * coef[:, :, None], dst, num_segments=N)
    out = out + h * coef_loop[:, :, None]
    return out.reshape(N, H * C) + bias


def kernel(x, edge_index, edge_attr, W1, aS1, aD1, We1, aE1, b1, g1, be1,
           W2, aS2, aD2, We2, aE2, b2, g2, be2):
    src, dst = edge_index[0], edge_index[1]
    h1 = _gat_layer(x, src, dst, edge_attr, W1, aS1, aD1, We1, aE1, b1, _HEADS, _HID)
    h1 = _ln(h1, g1, be1, relu=True)
    o = _gat_layer(h1, src, dst, edge_attr, W2, aS2, aD2, We2, aE2, b2, 1, _HID)
    return _ln(o, g2, be2, relu=False)


# speedup vs baseline: 3.6672x; 3.0591x over previous
import functools
import jax
import jax.numpy as jnp
from jax import lax
from jax.experimental import pallas as pl
from jax.experimental.pallas import tpu as pltpu
from jax.experimental.pallas import tpu_sc as plsc

_N = 10000
_E = 160000
_HID = 256
_HEADS = 4

_EB = 128
_CW = 128
_NB = 79
_EPAD = 16 * _NB * _EB
_NPAD = 10240
_RPT = _NPAD // 16


def _make_agg(nchunk, nheads):
    cpc = nchunk // 2
    mesh = plsc.VectorSubcoreMesh(core_axis_name="c", subcore_axis_name="s")

    @functools.partial(
        pl.kernel,
        mesh=mesh,
        out_type=jax.ShapeDtypeStruct((nchunk * _NPAD, _CW), jnp.float32),
        compiler_params=pltpu.CompilerParams(needs_layout_passes=False),
        scratch_types=[
            pltpu.VMEM((_EB,), jnp.int32),
            pltpu.VMEM((_EB,), jnp.int32),
            pltpu.VMEM((_EB,), jnp.int32),
            pltpu.VMEM((_EB,), jnp.float32),
            pltpu.VMEM((_EB, _CW), jnp.float32),
            pltpu.VMEM((_EB, _CW), jnp.float32),
            pltpu.VMEM_SHARED((_NPAD, _CW), jnp.float32),
            pltpu.SemaphoreType.DMA,
        ],
    )
    def agg(hch, srcp, dstp, coefp, out,
            srcb, srcjb, dstb, coefb, rows, zb, accum, sem):
        cid = lax.axis_index("c")
        sid = lax.axis_index("s")

        def zrow(i, c):
            for v in range(_CW // 16):
                zb[i, pl.ds(v * 16, 16)] = jnp.zeros((16,), jnp.float32)
            return c
        lax.fori_loop(0, _EB, zrow, 0)

        my0 = sid * _RPT
        for jj in range(cpc):
            j = cid * cpc + jj
            hd = j // (256 // _CW)
            jN = j * _NPAD
            for off in (0, 128, 256, 384, 512):
                pltpu.sync_copy(zb, accum.at[pl.ds(my0 + off, 128)])
            plsc.subcore_barrier()

            def batch(b, c):
                base = (sid * _NB + b) * _EB
                pltpu.sync_copy(srcp.at[pl.ds(base, _EB)], srcb)
                pltpu.sync_copy(dstp.at[pl.ds(base, _EB)], dstb)
                pltpu.sync_copy(coefp.at[pl.ds(hd * _EPAD + base, _EB)], coefb)

                def addj(i8, c2):
                    sl = pl.ds(i8 * 16, 16)
                    srcjb[sl] = srcb[sl] + jN
                    return c2
                lax.fori_loop(0, 8, addj, 0)

                pltpu.async_copy(hch.at[srcjb], rows, sem).wait()

                def scale(i, c2):
                    c16 = plsc.load_gather(coefb, [jnp.full((16,), i, jnp.int32)])
                    for v in range(_CW // 16):
                        sl = pl.ds(v * 16, 16)
                        rows[i, sl] = rows[i, sl] * c16
                    return c2
                lax.fori_loop(0, _EB, scale, 0)

                pltpu.sync_copy(rows, accum.at[dstb], add=True)
                return c
            lax.fori_loop(0, _NB, batch, 0)
            plsc.subcore_barrier()

            for off in (0, 128, 256, 384, 512):
                pltpu.sync_copy(accum.at[pl.ds(my0 + off, 128)], rows)
                pltpu.sync_copy(rows, out.at[pl.ds(jN + my0 + off, 128)])

    return agg


_agg1 = _make_agg(1024 // _CW, _HEADS)
_agg2 = _make_agg(256 // _CW, 1)


def _ln_relu_body(h_ref, g_ref, b_ref, o_ref, *, relu):
    h = h_ref[...]
    m = jnp.mean(h, axis=-1, keepdims=True)
    v = jnp.mean((h - m) ** 2, axis=-1, keepdims=True)
    y = (h - m) * jax.lax.rsqrt(v + 1e-5) * g_ref[...] + b_ref[...]
    if relu:
        y = jnp.maximum(y, 0.0)
    o_ref[...] = y


def _ln(h, g, b, relu):
    n, d = h.shape
    blk = 1000
    return pl.pallas_call(
        functools.partial(_ln_relu_body, relu=relu),
        out_shape=jax.ShapeDtypeStruct((n, d), jnp.float32),
        grid=(n // blk,),
        in_specs=[
            pl.BlockSpec((blk, d), lambda i: (i, 0)),
            pl.BlockSpec((1, d), lambda i: (0, 0)),
            pl.BlockSpec((1, d), lambda i: (0, 0)),
        ],
        out_specs=pl.BlockSpec((blk, d), lambda i: (i, 0)),
    )(h, g.reshape(1, d), b.reshape(1, d))


def _gat_layer(x, src, dst, ea, W, aS, aD, We, aE, bias, H, C, agg):
    N = x.shape[0]
    D = H * C
    nchunk = D // _CW
    h = (x @ W).reshape(N, H, C)
    a_src = (h * aS).sum(-1)
    a_dst = (h * aD).sum(-1)
    V = (We.reshape(We.shape[0], H, C) * aE).sum(-1)
    a_edge = ea @ V

    ones = jnp.ones((ea.shape[0],), jnp.float32)
    cnt = jax.ops.segment_sum(ones, dst, num_segments=N)
    ae_sum = jax.ops.segment_sum(a_edge, dst, num_segments=N)
    mean_ae = ae_sum / jnp.clip(cnt, 1.0)[:, None]

    alpha = jax.nn.leaky_relu(a_src[src] + a_dst[dst] + a_edge, 0.2)
    ex = jnp.exp(alpha)
    alpha_loop = jax.nn.leaky_relu(a_src + a_dst + mean_ae, 0.2)
    ex_loop = jnp.exp(alpha_loop)

    den = jax.ops.segment_sum(ex, dst, num_segments=N) + ex_loop
    coef_loop = ex_loop / (den + 1e-16)
    coef = ex / (den[dst] + 1e-16)

    if agg is None:
        out = jax.ops.segment_sum(h[src] * coef[:, :, None], dst, num_segments=N)
        out = out + h * coef_loop[:, :, None]
        return out.reshape(N, D) + bias
    hch = h.reshape(N, nchunk, _CW).transpose(1, 0, 2)
    hch = jnp.pad(hch, ((0, 0), (0, _NPAD - N), (0, 0))).reshape(nchunk * _NPAD, _CW)
    coefp = jnp.pad(coef.T, ((0, 0), (0, _EPAD - _E))).reshape(-1)
    srcp = jnp.pad(src.astype(jnp.int32), (0, _EPAD - _E))
    dstp = jnp.pad(dst.astype(jnp.int32), (0, _EPAD - _E))
    outc = agg(hch, srcp, dstp, coefp)
    out = outc.reshape(nchunk, _NPAD, _CW)[:, :N].transpose(1, 0, 2).reshape(N, H, C)

    out = out + h * coef_loop[:, :, None]
    return out.reshape(N, D) + bias


def kernel(x, edge_index, edge_attr, W1, aS1, aD1, We1, aE1, b1, g1, be1,
           W2, aS2, aD2, We2, aE2, b2, g2, be2):
    src, dst = edge_index[0], edge_index[1]
    h1 = _gat_layer(x, src, dst, edge_attr, W1, aS1, aD1, We1, aE1, b1,
                    _HEADS, _HID, _agg1)
    h1 = _ln(h1, g1, be1, relu=True)
    o = _gat_layer(h1, src, dst, edge_attr, W2, aS2, aD2, We2, aE2, b2,
                   1, _HID, _agg2)
    return _ln(o, g2, be2, relu=False)


# speedup vs baseline: 3.7613x; 1.0257x over previous
import functools
import jax
import jax.numpy as jnp
from jax import lax
from jax.experimental import pallas as pl
from jax.experimental.pallas import tpu as pltpu
from jax.experimental.pallas import tpu_sc as plsc

_N = 10000
_E = 160000
_HID = 256
_HEADS = 4

_EB = 128
_CW = 128
_NB = 80
_EPAD = 16 * _NB * _EB
_NPAD = 10240
_RPT = _NPAD // 16


def _make_agg(nchunk, nheads):
    cpc = nchunk // 2
    mesh = plsc.VectorSubcoreMesh(core_axis_name="c", subcore_axis_name="s")

    @functools.partial(
        pl.kernel,
        mesh=mesh,
        out_type=jax.ShapeDtypeStruct((nchunk * _NPAD, _CW), jnp.float32),
        compiler_params=pltpu.CompilerParams(needs_layout_passes=False),
        scratch_types=[
            pltpu.VMEM((_EB,), jnp.int32),
            pltpu.VMEM((_EB,), jnp.int32),
            pltpu.VMEM((_EB,), jnp.int32),
            pltpu.VMEM((_EB,), jnp.float32),
            pltpu.VMEM((_EB, _CW), jnp.float32),
            pltpu.VMEM((_EB,), jnp.int32),
            pltpu.VMEM((_EB,), jnp.int32),
            pltpu.VMEM((_EB,), jnp.int32),
            pltpu.VMEM((_EB,), jnp.float32),
            pltpu.VMEM((_EB, _CW), jnp.float32),
            pltpu.VMEM_SHARED((_NPAD, _CW), jnp.float32),
            pltpu.SemaphoreType.DMA,
            pltpu.SemaphoreType.DMA,
        ],
    )
    def agg(hch, srcp, dstp, coefp, out,
            srcbA, srcjbA, dstbA, coefbA, rowsA,
            srcbB, srcjbB, dstbB, coefbB, rowsB,
            accum, semA, semB):
        cid = lax.axis_index("c")
        sid = lax.axis_index("s")
        my0 = sid * _RPT

        for jj in range(cpc):
            j = cid * cpc + jj
            hd = j // (256 // _CW)
            jN = j * _NPAD

            bufA = (srcbA, srcjbA, dstbA, coefbA, rowsA, semA)
            bufB = (srcbB, srcjbB, dstbB, coefbB, rowsB, semB)

            def stage_fire(b, buf):
                srcb, srcjb, dstb, coefb, rows, sem = buf
                base = (sid * _NB + b) * _EB
                pltpu.sync_copy(srcp.at[pl.ds(base, _EB)], srcb)
                pltpu.sync_copy(dstp.at[pl.ds(base, _EB)], dstb)
                pltpu.sync_copy(coefp.at[pl.ds(hd * _EPAD + base, _EB)], coefb)

                def addj(i8, c2):
                    sl = pl.ds(i8 * 16, 16)
                    srcjb[sl] = srcb[sl] + jN
                    return c2
                lax.fori_loop(0, _EB // 16, addj, 0)
                pltpu.async_copy(hch.at[srcjb], rows, sem)

            def process(buf):
                srcb, srcjb, dstb, coefb, rows, sem = buf
                pltpu.make_async_copy(hch.at[srcjb], rows, sem).wait()

                def scale(i, c2):
                    c16 = plsc.load_gather(coefb, [jnp.full((16,), i, jnp.int32)])
                    for v in range(_CW // 16):
                        sl = pl.ds(v * 16, 16)
                        rows[i, sl] = rows[i, sl] * c16
                    return c2
                lax.fori_loop(0, _EB, scale, 0)
                pltpu.sync_copy(rows, accum.at[dstb], add=True)

            def zrow(i, c):
                for v in range(_CW // 16):
                    rowsB[i, pl.ds(v * 16, 16)] = jnp.zeros((16,), jnp.float32)
                return c
            lax.fori_loop(0, _EB, zrow, 0)
            for off in (0, 128, 256, 384, 512):
                pltpu.sync_copy(rowsB, accum.at[pl.ds(my0 + off, 128)])
            plsc.subcore_barrier()

            stage_fire(0, bufA)

            def pair(i, c):
                stage_fire(2 * i + 1, bufB)
                process(bufA)

                @pl.when(i < _NB // 2 - 1)
                def _():
                    stage_fire(2 * i + 2, bufA)
                process(bufB)
                return c
            lax.fori_loop(0, _NB // 2, pair, 0)
            plsc.subcore_barrier()

            for off in (0, 128, 256, 384, 512):
                pltpu.sync_copy(accum.at[pl.ds(my0 + off, 128)], rowsA)
                pltpu.sync_copy(rowsA, out.at[pl.ds(jN + my0 + off, 128)])

    return agg


_agg1 = _make_agg(1024 // _CW, _HEADS)
_agg2 = _make_agg(256 // _CW, 1)


def _ln_relu_body(h_ref, g_ref, b_ref, o_ref, *, relu):
    h = h_ref[...]
    m = jnp.mean(h, axis=-1, keepdims=True)
    v = jnp.mean((h - m) ** 2, axis=-1, keepdims=True)
    y = (h - m) * jax.lax.rsqrt(v + 1e-5) * g_ref[...] + b_ref[...]
    if relu:
        y = jnp.maximum(y, 0.0)
    o_ref[...] = y


def _ln(h, g, b, relu):
    n, d = h.shape
    blk = 1000
    return pl.pallas_call(
        functools.partial(_ln_relu_body, relu=relu),
        out_shape=jax.ShapeDtypeStruct((n, d), jnp.float32),
        grid=(n // blk,),
        in_specs=[
            pl.BlockSpec((blk, d), lambda i: (i, 0)),
            pl.BlockSpec((1, d), lambda i: (0, 0)),
            pl.BlockSpec((1, d), lambda i: (0, 0)),
        ],
        out_specs=pl.BlockSpec((blk, d), lambda i: (i, 0)),
    )(h, g.reshape(1, d), b.reshape(1, d))


def _gat_layer(x, src, dst, ea, W, aS, aD, We, aE, bias, H, C, agg):
    N = x.shape[0]
    D = H * C
    nchunk = D // _CW
    h = (x @ W).reshape(N, H, C)
    a_src = (h * aS).sum(-1)
    a_dst = (h * aD).sum(-1)
    V = (We.reshape(We.shape[0], H, C) * aE).sum(-1)
    a_edge = ea @ V

    ones = jnp.ones((ea.shape[0],), jnp.float32)
    cnt = jax.ops.segment_sum(ones, dst, num_segments=N)
    ae_sum = jax.ops.segment_sum(a_edge, dst, num_segments=N)
    mean_ae = ae_sum / jnp.clip(cnt, 1.0)[:, None]

    alpha = jax.nn.leaky_relu(a_src[src] + a_dst[dst] + a_edge, 0.2)
    ex = jnp.exp(alpha)
    alpha_loop = jax.nn.leaky_relu(a_src + a_dst + mean_ae, 0.2)
    ex_loop = jnp.exp(alpha_loop)

    den = jax.ops.segment_sum(ex, dst, num_segments=N) + ex_loop
    coef_loop = ex_loop / (den + 1e-16)
    coef = ex / (den[dst] + 1e-16)

    hch = h.reshape(N, nchunk, _CW).transpose(1, 0, 2)
    hch = jnp.pad(hch, ((0, 0), (0, _NPAD - N), (0, 0))).reshape(nchunk * _NPAD, _CW)
    coefp = jnp.pad(coef.T, ((0, 0), (0, _EPAD - _E))).reshape(-1)
    srcp = jnp.pad(src.astype(jnp.int32), (0, _EPAD - _E))
    dstp = jnp.pad(dst.astype(jnp.int32), (0, _EPAD - _E))
    outc = agg(hch, srcp, dstp, coefp)
    out = outc.reshape(nchunk, _NPAD, _CW)[:, :N].transpose(1, 0, 2).reshape(N, H, C)

    out = out + h * coef_loop[:, :, None]
    return out.reshape(N, D) + bias


def kernel(x, edge_index, edge_attr, W1, aS1, aD1, We1, aE1, b1, g1, be1,
           W2, aS2, aD2, We2, aE2, b2, g2, be2):
    src, dst = edge_index[0], edge_index[1]
    h1 = _gat_layer(x, src, dst, edge_attr, W1, aS1, aD1, We1, aE1, b1,
                    _HEADS, _HID, _agg1)
    h1 = _ln(h1, g1, be1, relu=True)
    o = _gat_layer(h1, src, dst, edge_attr, W2, aS2, aD2, We2, aE2, b2,
                   1, _HID, _agg2)
    return _ln(o, g2, be2, relu=False)


# speedup vs baseline: 4.2857x; 1.1394x over previous
import functools
import jax
import jax.numpy as jnp
from jax import lax
from jax.experimental import pallas as pl
from jax.experimental.pallas import tpu as pltpu
from jax.experimental.pallas import tpu_sc as plsc

_N = 10000
_E = 160000
_HID = 256
_HEADS = 4

_EB = 128
_CW = 128
_NB = 80
_EPAD = 16 * _NB * _EB
_NPAD = 10240
_RPT = _NPAD // 16


def _make_agg(nchunk, nheads):
    cpc = nchunk // 2
    mesh = plsc.VectorSubcoreMesh(core_axis_name="c", subcore_axis_name="s")

    @functools.partial(
        pl.kernel,
        mesh=mesh,
        out_type=jax.ShapeDtypeStruct((nchunk * _NPAD, _CW), jnp.float32),
        compiler_params=pltpu.CompilerParams(needs_layout_passes=False),
        scratch_types=[
            pltpu.VMEM((_EB,), jnp.int32),
            pltpu.VMEM((_EB,), jnp.int32),
            pltpu.VMEM((_EB,), jnp.int32),
            pltpu.VMEM((_EB,), jnp.float32),
            pltpu.VMEM((_EB, _CW), jnp.float32),
            pltpu.VMEM((_EB,), jnp.int32),
            pltpu.VMEM((_EB,), jnp.int32),
            pltpu.VMEM((_EB,), jnp.int32),
            pltpu.VMEM((_EB,), jnp.float32),
            pltpu.VMEM((_EB, _CW), jnp.float32),
            pltpu.VMEM_SHARED((_NPAD, _CW), jnp.float32),
            pltpu.SemaphoreType.DMA,
            pltpu.SemaphoreType.DMA,
        ],
    )
    def agg(hch, srcp, dstp, coefp, out,
            srcbA, srcjbA, dstbA, coefbA, rowsA,
            srcbB, srcjbB, dstbB, coefbB, rowsB,
            accum, semA, semB):
        cid = lax.axis_index("c")
        sid = lax.axis_index("s")
        my0 = sid * _RPT

        for jj in range(cpc):
            j = cid * cpc + jj
            hd = j // (256 // _CW)
            jN = j * _NPAD

            bufA = (srcbA, srcjbA, dstbA, coefbA, rowsA, semA)
            bufB = (srcbB, srcjbB, dstbB, coefbB, rowsB, semB)

            def stage_fire(b, buf):
                srcb, srcjb, dstb, coefb, rows, sem = buf
                base = (sid * _NB + b) * _EB
                pltpu.sync_copy(srcp.at[pl.ds(base, _EB)], srcb)
                pltpu.sync_copy(dstp.at[pl.ds(base, _EB)], dstb)
                pltpu.sync_copy(coefp.at[pl.ds(hd * _EPAD + base, _EB)], coefb)

                def addj(i8, c2):
                    sl = pl.ds(i8 * 16, 16)
                    srcjb[sl] = srcb[sl] + jN
                    return c2
                lax.fori_loop(0, _EB // 16, addj, 0)
                pltpu.async_copy(hch.at[srcjb], rows, sem)

            def process(buf):
                srcb, srcjb, dstb, coefb, rows, sem = buf
                pltpu.make_async_copy(hch.at[srcjb], rows, sem).wait()

                def scale(i, c2):
                    c16 = plsc.load_gather(coefb, [jnp.full((16,), i, jnp.int32)])
                    for v in range(_CW // 16):
                        sl = pl.ds(v * 16, 16)
                        rows[i, sl] = rows[i, sl] * c16
                    return c2
                lax.fori_loop(0, _EB, scale, 0)
                pltpu.sync_copy(rows, accum.at[dstb], add=True)

            def zrow(i, c):
                for v in range(_CW // 16):
                    rowsB[i, pl.ds(v * 16, 16)] = jnp.zeros((16,), jnp.float32)
                return c
            lax.fori_loop(0, _EB, zrow, 0)
            for off in (0, 128, 256, 384, 512):
                pltpu.sync_copy(rowsB, accum.at[pl.ds(my0 + off, 128)])
            plsc.subcore_barrier()

            stage_fire(0, bufA)

            def pair(i, c):
                stage_fire(2 * i + 1, bufB)
                process(bufA)

                @pl.when(i < _NB // 2 - 1)
                def _():
                    stage_fire(2 * i + 2, bufA)
                process(bufB)
                return c
            lax.fori_loop(0, _NB // 2, pair, 0)
            plsc.subcore_barrier()

            for off in (0, 128, 256, 384, 512):
                pltpu.sync_copy(accum.at[pl.ds(my0 + off, 128)], rowsA)
                pltpu.sync_copy(rowsA, out.at[pl.ds(jN + my0 + off, 128)])

    return agg


_agg1 = _make_agg(1024 // _CW, _HEADS)
_agg2 = _make_agg(256 // _CW, 1)


def _ln_relu_body(h_ref, g_ref, b_ref, o_ref, *, relu):
    h = h_ref[...]
    m = jnp.mean(h, axis=-1, keepdims=True)
    v = jnp.mean((h - m) ** 2, axis=-1, keepdims=True)
    y = (h - m) * jax.lax.rsqrt(v + 1e-5) * g_ref[...] + b_ref[...]
    if relu:
        y = jnp.maximum(y, 0.0)
    o_ref[...] = y


def _ln(h, g, b, relu):
    n, d = h.shape
    blk = 1000
    return pl.pallas_call(
        functools.partial(_ln_relu_body, relu=relu),
        out_shape=jax.ShapeDtypeStruct((n, d), jnp.float32),
        grid=(n // blk,),
        in_specs=[
            pl.BlockSpec((blk, d), lambda i: (i, 0)),
            pl.BlockSpec((1, d), lambda i: (0, 0)),
            pl.BlockSpec((1, d), lambda i: (0, 0)),
        ],
        out_specs=pl.BlockSpec((blk, d), lambda i: (i, 0)),
    )(h, g.reshape(1, d), b.reshape(1, d))


def _gat_layer(x, src, dst, ea, W, aS, aD, We, aE, bias, H, C, agg):
    N = x.shape[0]
    D = H * C
    nchunk = D // _CW
    h = (x @ W).reshape(N, H, C)
    a_src = (h * aS).sum(-1)
    a_dst = (h * aD).sum(-1)
    V = (We.reshape(We.shape[0], H, C) * aE).sum(-1)
    a_edge = ea @ V

    ones = jnp.ones((ea.shape[0], 1), jnp.float32)
    alpha = jax.nn.leaky_relu(a_src[src] + a_dst[dst] + a_edge, 0.2)
    ex = jnp.exp(alpha)
    packed = jax.ops.segment_sum(
        jnp.concatenate([ones, a_edge, ex], axis=1), dst, num_segments=N)
    cnt, ae_sum, ex_sum = packed[:, 0], packed[:, 1:1 + H], packed[:, 1 + H:]
    mean_ae = ae_sum / jnp.clip(cnt, 1.0)[:, None]

    alpha_loop = jax.nn.leaky_relu(a_src + a_dst + mean_ae, 0.2)
    ex_loop = jnp.exp(alpha_loop)

    den = ex_sum + ex_loop
    coef_loop = ex_loop / (den + 1e-16)
    coef = ex / (den[dst] + 1e-16)

    hch = h.reshape(N, nchunk, _CW).transpose(1, 0, 2)
    hch = jnp.pad(hch, ((0, 0), (0, _NPAD - N), (0, 0))).reshape(nchunk * _NPAD, _CW)
    coefp = jnp.pad(coef.T, ((0, 0), (0, _EPAD - _E))).reshape(-1)
    srcp = jnp.pad(src.astype(jnp.int32), (0, _EPAD - _E))
    dstp = jnp.pad(dst.astype(jnp.int32), (0, _EPAD - _E))
    outc = agg(hch, srcp, dstp, coefp)
    out = outc.reshape(nchunk, _NPAD, _CW)[:, :N].transpose(1, 0, 2).reshape(N, H, C)

    out = out + h * coef_loop[:, :, None]
    return out.reshape(N, D) + bias


def kernel(x, edge_index, edge_attr, W1, aS1, aD1, We1, aE1, b1, g1, be1,
           W2, aS2, aD2, We2, aE2, b2, g2, be2):
    src, dst = edge_index[0], edge_index[1]
    h1 = _gat_layer(x, src, dst, edge_attr, W1, aS1, aD1, We1, aE1, b1,
                    _HEADS, _HID, _agg1)
    h1 = _ln(h1, g1, be1, relu=True)
    o = _gat_layer(h1, src, dst, edge_attr, W2, aS2, aD2, We2, aE2, b2,
                   1, _HID, _agg2)
    return _ln(o, g2, be2, relu=False)
